# baseline (speedup 1.0000x reference)
import functools

import jax
import jax.numpy as jnp
from jax import lax
from jax.experimental import pallas as pl
from jax.experimental.pallas import tpu as pltpu
from jax.experimental.pallas import tpu_sc as plsc

N = 10000
E = 320000
HID = 128
HEADS = 4
OC = 32

NC, NS, LANES = 2, 16, 16
NW = NC * NS
CH = 128
NPAD = 10240
RPT = NPAD // NS
EPW = 10240
CPW = EPW // CH
EPAD = NW * EPW
CPW2 = 2 * CPW
IB = 16
EW8 = EPAD // 8
RPC = CH // 8


@functools.cache
def _mesh():
    return plsc.VectorSubcoreMesh(
        core_axis_name="c", subcore_axis_name="s",
        num_cores=NC, num_subcores=NS)

f32 = jnp.float32
i32 = jnp.int32


def _ids():
    cid = lax.axis_index("c")
    sid = lax.axis_index("s")
    return cid, sid, sid * NC + cid


def _zero_acc(z_hbm, acc, sid):
    r0 = sid * RPT
    pltpu.sync_copy(z_hbm.at[pl.ds(r0, RPT)], acc.at[pl.ds(r0, RPT)])
    plsc.subcore_barrier()


def _dump_acc(acc, out_hbm, cid, sid):
    plsc.subcore_barrier()
    r0 = sid * RPT
    pltpu.sync_copy(acc.at[pl.ds(r0, RPT)], out_hbm.at[cid, pl.ds(r0, RPT)])


def _sc_deg():
    def body(d_hbm, ones_hbm, z_hbm, out_hbm, didx, onesb, s0, s1, acc):
        cid, sid, wid = _ids()
        pltpu.sync_copy(ones_hbm, onesb)
        _zero_acc(z_hbm, acc, sid)
        ssems = (s0, s1)

        def blk(k, _):
            pltpu.sync_copy(d_hbm.at[wid, pl.ds(k * IB, IB)], didx)
            for b in range(2):
                pltpu.async_copy(onesb, acc.at[didx.at[b]], ssems[b],
                                 add=True)

            def step(g, _):
                for b in range(2):
                    c = g * 2 + b
                    pltpu.make_async_copy(
                        onesb, acc.at[didx.at[c]], ssems[b]).wait()

                    @pl.when(c + 2 < IB)
                    def _():
                        pltpu.async_copy(
                            onesb, acc.at[didx.at[c + 2]], ssems[b], add=True)
                return ()
            lax.fori_loop(0, IB // 2, step, ())
            return ()
        lax.fori_loop(0, CPW // IB, blk, ())
        _dump_acc(acc, out_hbm, cid, sid)

    return pl.kernel(
        body,
        out_type=jax.ShapeDtypeStruct((NC, NPAD, LANES), f32),
        mesh=_mesh(),
        scratch_types=[
            pltpu.VMEM((IB, CH), i32),
            pltpu.VMEM((CH, LANES), f32),
            pltpu.SemaphoreType.DMA,
            pltpu.SemaphoreType.DMA,
            pltpu.VMEM_SHARED((NPAD, LANES), f32),
        ],
    )


def _sc_agg():
    def body(s_hbm, d_hbm, y_hbm, z_hbm, out_hbm,
             sidx, didx, b0, b1, g0a, g0b, g1a, g1b, s0, s1, acc):
        cid, sid, wid = _ids()
        _zero_acc(z_hbm, acc, sid)

        bufs = (b0, b1)
        gsems = ((g0a, g0b), (g1a, g1b))
        ssems = (s0, s1)
        H = CH // 2

        def start_g(c, b):
            pltpu.async_copy(y_hbm.at[sidx.at[c, pl.ds(0, H)]],
                             bufs[b].at[pl.ds(0, H)], gsems[b][0])
            pltpu.async_copy(y_hbm.at[sidx.at[c, pl.ds(H, H)]],
                             bufs[b].at[pl.ds(H, H)], gsems[b][1])

        def wait_g(c, b):
            pltpu.make_async_copy(y_hbm.at[sidx.at[c, pl.ds(0, H)]],
                                  bufs[b].at[pl.ds(0, H)], gsems[b][0]).wait()
            pltpu.make_async_copy(y_hbm.at[sidx.at[c, pl.ds(H, H)]],
                                  bufs[b].at[pl.ds(H, H)], gsems[b][1]).wait()

        def blk(k, _):
            pltpu.sync_copy(s_hbm.at[wid, pl.ds(k * IB, IB)], sidx)
            pltpu.sync_copy(d_hbm.at[wid, pl.ds(k * IB, IB)], didx)
            for b in range(2):
                start_g(b, b)

            def step(g, _):
                for b in range(2):
                    c = g * 2 + b
                    wait_g(c, b)
                    pltpu.async_copy(
                        bufs[b], acc.at[didx.at[c]], ssems[b], add=True)
                    pltpu.make_async_copy(
                        bufs[b], acc.at[didx.at[c]], ssems[b]).wait()

                    @pl.when(c + 2 < IB)
                    def _():
                        start_g(c + 2, b)
                return ()
            lax.fori_loop(0, IB // 2, step, ())
            return ()
        lax.fori_loop(0, CPW // IB, blk, ())
        _dump_acc(acc, out_hbm, cid, sid)

    return pl.kernel(
        body,
        out_type=jax.ShapeDtypeStruct((NC, NPAD, HID), f32),
        mesh=_mesh(),
        scratch_types=[
            pltpu.VMEM((IB, CH), i32),
            pltpu.VMEM((IB, CH), i32),
            pltpu.VMEM((CH, HID), f32),
            pltpu.VMEM((CH, HID), f32),
            pltpu.SemaphoreType.DMA,
            pltpu.SemaphoreType.DMA,
            pltpu.SemaphoreType.DMA,
            pltpu.SemaphoreType.DMA,
            pltpu.SemaphoreType.DMA,
            pltpu.SemaphoreType.DMA,
            pltpu.VMEM_SHARED((NPAD, HID), f32),
        ],
    )


def _sc_recgather():
    def body(sd_hbm, rec_hbm, out_hbm, idx, b0, b1, p0, p1,
             g0, g1, w0, w1, tbl):
        cid, sid, wid = _ids()
        r0 = sid * RPT
        pltpu.sync_copy(rec_hbm.at[pl.ds(r0, RPT)], tbl.at[pl.ds(r0, RPT)])
        plsc.subcore_barrier()

        bufs = (b0, b1)
        pbufs = (p0, p1)
        gsems = (g0, g1)
        wsems = (w0, w1)

        def blk(k, _):
            pltpu.sync_copy(sd_hbm.at[wid, pl.ds(k * IB, IB)], idx)
            for b in range(2):
                pltpu.async_copy(tbl.at[idx.at[b]], bufs[b], gsems[b])

            def step(g, _):
                for b in range(2):
                    c = g * 2 + b
                    tg = k * IB + c
                    off = ((tg // CPW) * EW8 + wid * (EPW // 8)
                           + (tg % CPW) * RPC)
                    pltpu.make_async_copy(
                        tbl.at[idx.at[c]], bufs[b], gsems[b]).wait()

                    buf, pbuf = bufs[b], pbufs[b]
                    for r in range(RPC):
                        for q in range(8):
                            pbuf[r, pl.ds(q * 16, 16)] = (
                                buf[r * 8 + q, pl.ds(0, 16)])
                    pltpu.async_copy(
                        pbuf, out_hbm.at[pl.ds(off, RPC)], wsems[b])
                    pltpu.make_async_copy(
                        pbuf, out_hbm.at[pl.ds(off, RPC)], wsems[b]).wait()

                    @pl.when(c + 2 < IB)
                    def _():
                        pltpu.async_copy(
                            tbl.at[idx.at[c + 2]], bufs[b], gsems[b])
                return ()
            lax.fori_loop(0, IB // 2, step, ())
            return ()
        lax.fori_loop(0, CPW2 // IB, blk, ())

    return pl.kernel(
        body,
        out_type=jax.ShapeDtypeStruct((2 * EW8, HID), f32),
        mesh=_mesh(),
        scratch_types=[
            pltpu.VMEM((IB, CH), i32),
            pltpu.VMEM((CH, LANES), f32),
            pltpu.VMEM((CH, LANES), f32),
            pltpu.VMEM((RPC, HID), f32),
            pltpu.VMEM((RPC, HID), f32),
            pltpu.SemaphoreType.DMA,
            pltpu.SemaphoreType.DMA,
            pltpu.SemaphoreType.DMA,
            pltpu.SemaphoreType.DMA,
            pltpu.VMEM_SHARED((NPAD, LANES), f32),
        ],
    )


def _sc_gat():
    def body(s_hbm, d_hbm, xwr_hbm, exw_hbm, z_hbm, out_hbm,
             sidx, didx, gb0, gb1, eb0, eb1,
             g0, g1, s0, s1, e0, e1, acc):
        cid, sid, wid = _ids()
        _zero_acc(z_hbm, acc, sid)

        gbufs = (gb0, gb1)
        ebufs = (eb0, eb1)
        gsems = (g0, g1)
        ssems = (s0, s1)
        esems = (e0, e1)
        erow0 = wid * (EPW // 8)

        def blk(k, _):
            pltpu.sync_copy(s_hbm.at[wid, pl.ds(k * IB, IB)], sidx)
            pltpu.sync_copy(d_hbm.at[wid, pl.ds(k * IB, IB)], didx)
            for b in range(2):
                pltpu.async_copy(xwr_hbm.at[sidx.at[b]], gbufs[b], gsems[b])
                pltpu.async_copy(
                    exw_hbm.at[pl.ds(erow0 + (k * IB + b) * RPC, RPC)],
                    ebufs[b], esems[b])

            def step(g, _):
                for b in range(2):
                    c = g * 2 + b
                    pltpu.make_async_copy(
                        xwr_hbm.at[sidx.at[c]], gbufs[b], gsems[b]).wait()
                    pltpu.make_async_copy(
                        exw_hbm.at[pl.ds(erow0 + (k * IB + c) * RPC, RPC)],
                        ebufs[b], esems[b]).wait()

                    gbuf, ebuf = gbufs[b], ebufs[b]

                    def edge(i, _):
                        exv = ebuf[i >> 3, pl.ds((i & 7) * 16, 16)]
                        for j in range(HID // LANES):
                            gbuf[i, pl.ds(j * LANES, LANES)] = (
                                gbuf[i, pl.ds(j * LANES, LANES)] * exv)
                        return ()
                    lax.fori_loop(0, CH, edge, (), unroll=4)

                    pltpu.async_copy(
                        gbuf, acc.at[didx.at[c]], ssems[b], add=True)
                    pltpu.make_async_copy(
                        gbuf, acc.at[didx.at[c]], ssems[b]).wait()

                    @pl.when(c + 2 < IB)
                    def _():
                        pltpu.async_copy(
                            xwr_hbm.at[sidx.at[c + 2]], gbufs[b], gsems[b])
                        pltpu.async_copy(
                            exw_hbm.at[
                                pl.ds(erow0 + (k * IB + c + 2) * RPC, RPC)],
                            ebufs[b], esems[b])
                return ()
            lax.fori_loop(0, IB // 2, step, ())
            return ()
        lax.fori_loop(0, CPW // IB, blk, ())
        _dump_acc(acc, out_hbm, cid, sid)

    return pl.kernel(
        body,
        out_type=jax.ShapeDtypeStruct((NC, NPAD, HID), f32),
        mesh=_mesh(),
        scratch_types=[
            pltpu.VMEM((IB, CH), i32),
            pltpu.VMEM((IB, CH), i32),
            pltpu.VMEM((CH, HID), f32),
            pltpu.VMEM((CH, HID), f32),
            pltpu.VMEM((RPC, HID), f32),
            pltpu.VMEM((RPC, HID), f32),
            pltpu.SemaphoreType.DMA,
            pltpu.SemaphoreType.DMA,
            pltpu.SemaphoreType.DMA,
            pltpu.SemaphoreType.DMA,
            pltpu.SemaphoreType.DMA,
            pltpu.SemaphoreType.DMA,
            pltpu.VMEM_SHARED((NPAD, HID), f32),
        ],
    )


def _sc_den():
    def body(d_hbm, exw_hbm, z_hbm, out_hbm, didx, w0, w1, sb,
             l0, l1, s0, s1, acc):
        cid, sid, wid = _ids()
        _zero_acc(z_hbm, acc, sid)

        wbufs = (w0, w1)
        lsems = (l0, l1)
        ssems = (s0, s1)
        erow0 = wid * (EPW // 8)

        def blk(k, _):
            pltpu.sync_copy(d_hbm.at[wid, pl.ds(k * IB, IB)], didx)
            for b in range(2):
                pltpu.async_copy(
                    exw_hbm.at[pl.ds(erow0 + (k * IB + b) * RPC, RPC)],
                    wbufs[b], lsems[b])

            def step(g, _):
                for b in range(2):
                    c = g * 2 + b
                    pltpu.make_async_copy(
                        exw_hbm.at[pl.ds(erow0 + (k * IB + c) * RPC, RPC)],
                        wbufs[b], lsems[b]).wait()
                    wbuf = wbufs[b]
                    for r in range(RPC):
                        for q in range(8):
                            sb[r * 8 + q, pl.ds(0, 16)] = (
                                wbuf[r, pl.ds(q * 16, 16)])
                    pltpu.async_copy(
                        sb, acc.at[didx.at[c]], ssems[b], add=True)
                    pltpu.make_async_copy(
                        sb, acc.at[didx.at[c]], ssems[b]).wait()

                    @pl.when(c + 2 < IB)
                    def _():
                        pltpu.async_copy(
                            exw_hbm.at[
                                pl.ds(erow0 + (k * IB + c + 2) * RPC, RPC)],
                            wbufs[b], lsems[b])
                return ()
            lax.fori_loop(0, IB // 2, step, ())
            return ()
        lax.fori_loop(0, CPW // IB, blk, ())
        _dump_acc(acc, out_hbm, cid, sid)

    return pl.kernel(
        body,
        out_type=jax.ShapeDtypeStruct((NC, NPAD, LANES), f32),
        mesh=_mesh(),
        scratch_types=[
            pltpu.VMEM((IB, CH), i32),
            pltpu.VMEM((RPC, HID), f32),
            pltpu.VMEM((RPC, HID), f32),
            pltpu.VMEM((CH, LANES), f32),
            pltpu.SemaphoreType.DMA,
            pltpu.SemaphoreType.DMA,
            pltpu.SemaphoreType.DMA,
            pltpu.SemaphoreType.DMA,
            pltpu.VMEM_SHARED((NPAD, LANES), f32),
        ],
    )


BN = 1024
NB = NPAD // BN
BGE = 2048
NBG = EW8 // BGE


def _leaky(z):
    return jnp.maximum(z, 0.2 * z)


def _tc_pre(x_p, W1, degp):
    def body(x_ref, w_ref, dp_ref, dinv_ref, y_ref):
        d2 = dp_ref[0] + dp_ref[1]
        deg = d2[:, 0:1] + 1.0
        dinv = lax.rsqrt(deg)
        dinv_ref[...] = jnp.broadcast_to(dinv, (BN, HID))
        y_ref[...] = dinv * jnp.dot(x_ref[...], w_ref[...],
                                    preferred_element_type=f32)
    return pl.pallas_call(
        body,
        grid=(NB,),
        in_specs=[
            pl.BlockSpec((BN, HID), lambda i: (i, 0)),
            pl.BlockSpec((HID, HID), lambda i: (0, 0)),
            pl.BlockSpec((NC, BN, LANES), lambda i: (0, i, 0)),
        ],
        out_specs=[
            pl.BlockSpec((BN, HID), lambda i: (i, 0)),
            pl.BlockSpec((BN, HID), lambda i: (i, 0)),
        ],
        out_shape=[
            jax.ShapeDtypeStruct((NPAD, HID), f32),
            jax.ShapeDtypeStruct((NPAD, HID), f32),
        ],
    )(x_p, W1, degp)


def _tc_layer1(parts, y1, dinvf, b1, W2):
    def body(p_ref, y_ref, di_ref, b_ref, w_ref, h_ref, yn_ref):
        di = di_ref[...]
        h = jnp.maximum(
            di * (p_ref[0] + p_ref[1] + y_ref[...]) + b_ref[...], 0.0)
        h_ref[...] = h
        yn_ref[...] = di * jnp.dot(h, w_ref[...], preferred_element_type=f32)
    return pl.pallas_call(
        body,
        grid=(NB,),
        in_specs=[
            pl.BlockSpec((NC, BN, HID), lambda i: (0, i, 0)),
            pl.BlockSpec((BN, HID), lambda i: (i, 0)),
            pl.BlockSpec((BN, HID), lambda i: (i, 0)),
            pl.BlockSpec((1, HID), lambda i: (0, 0)),
            pl.BlockSpec((HID, HID), lambda i: (0, 0)),
        ],
        out_specs=[
            pl.BlockSpec((BN, HID), lambda i: (i, 0)),
            pl.BlockSpec((BN, HID), lambda i: (i, 0)),
        ],
        out_shape=[
            jax.ShapeDtypeStruct((NPAD, HID), f32),
            jax.ShapeDtypeStruct((NPAD, HID), f32),
        ],
    )(parts, y1, dinvf, b1, W2)


def _tc_layer2(parts, y2, dinvf, b2, h1, W3):
    def body(p_ref, y_ref, di_ref, b_ref, h1_ref, w_ref, yn_ref):
        di = di_ref[...]
        h = jnp.maximum(
            di * (p_ref[0] + p_ref[1] + y_ref[...]) + b_ref[...],
            0.0) + h1_ref[...]
        yn = di[:, : HID // 2] * jnp.dot(h, w_ref[...],
                                         preferred_element_type=f32)
        yn_ref[...] = jnp.concatenate(
            [yn, jnp.zeros((BN, HID // 2), f32)], axis=1)
    return pl.pallas_call(
        body,
        grid=(NB,),
        in_specs=[
            pl.BlockSpec((NC, BN, HID), lambda i: (0, i, 0)),
            pl.BlockSpec((BN, HID), lambda i: (i, 0)),
            pl.BlockSpec((BN, HID), lambda i: (i, 0)),
            pl.BlockSpec((1, HID), lambda i: (0, 0)),
            pl.BlockSpec((BN, HID), lambda i: (i, 0)),
            pl.BlockSpec((HID, HID // 2), lambda i: (0, 0)),
        ],
        out_specs=[pl.BlockSpec((BN, HID), lambda i: (i, 0))],
        out_shape=[jax.ShapeDtypeStruct((NPAD, HID), f32)],
    )(parts, y2, dinvf, b2, h1, W3)


def _tc_layer3(parts, y3p, dinvf, b3, Wa, P, Arec):
    def body(p_ref, y_ref, di_ref, b_ref, wa_ref, pm_ref, ar_ref,
             xwr_ref, rec_ref):
        H2 = HID // 2
        a = (p_ref[0] + p_ref[1] + y_ref[...])[:, :H2]
        di = di_ref[...][:, :H2]
        h = jnp.maximum(di * a + b_ref[...], 0.0)
        xw = jnp.dot(h, wa_ref[...], preferred_element_type=f32)
        xwr_ref[...] = jnp.dot(xw, pm_ref[...], preferred_element_type=f32)
        rec_ref[...] = jnp.dot(xw, ar_ref[...], preferred_element_type=f32)
    return pl.pallas_call(
        body,
        grid=(NB,),
        in_specs=[
            pl.BlockSpec((NC, BN, HID), lambda i: (0, i, 0)),
            pl.BlockSpec((BN, HID), lambda i: (i, 0)),
            pl.BlockSpec((BN, HID), lambda i: (i, 0)),
            pl.BlockSpec((1, HID // 2), lambda i: (0, 0)),
            pl.BlockSpec((HID // 2, HID), lambda i: (0, 0)),
            pl.BlockSpec((HID, HID), lambda i: (0, 0)),
            pl.BlockSpec((HID, LANES), lambda i: (0, 0)),
        ],
        out_specs=[
            pl.BlockSpec((BN, HID), lambda i: (i, 0)),
            pl.BlockSpec((BN, LANES), lambda i: (i, 0)),
        ],
        out_shape=[
            jax.ShapeDtypeStruct((NPAD, HID), f32),
            jax.ShapeDtypeStruct((NPAD, LANES), f32),
        ],
    )(parts, y3p, dinvf, b3, Wa, P, Arec)


def _tc_maxrec(rec):
    def body(r_ref, m_ref):
        i = pl.program_id(0)
        bm = jnp.max(r_ref[...], axis=0, keepdims=True)

        @pl.when(i == 0)
        def _():
            m_ref[...] = bm

        @pl.when(i > 0)
        def _():
            m_ref[...] = jnp.maximum(m_ref[...], bm)
    return pl.pallas_call(
        body,
        grid=(NB,),
        in_specs=[pl.BlockSpec((BN, LANES), lambda i: (i, 0))],
        out_specs=pl.BlockSpec((1, LANES), lambda i: (0, 0)),
        out_shape=jax.ShapeDtypeStruct((1, LANES), f32),
    )(rec)


def _tc_exp(gsd, M):
    def body(gs_ref, gd_ref, m_ref, ex_ref):
        gs = gs_ref[...]
        gd = gd_ref[...]
        m4 = m_ref[...][:, 0:4]
        cols = []
        for q in range(8):
            as4 = gs[:, 16 * q:16 * q + 4]
            ad4 = gd[:, 16 * q + 4:16 * q + 8]
            ex = jnp.exp(_leaky(as4 + ad4) - _leaky(ad4 + m4))
            cols.append(jnp.concatenate([ex, ex, ex, ex], axis=1))
        ex_ref[...] = jnp.concatenate(cols, axis=1)
    return pl.pallas_call(
        body,
        grid=(NBG,),
        in_specs=[
            pl.BlockSpec((BGE, HID), lambda i: (i, 0)),
            pl.BlockSpec((BGE, HID), lambda i: (i + NBG, 0)),
            pl.BlockSpec((1, LANES), lambda i: (0, 0)),
        ],
        out_specs=pl.BlockSpec((BGE, HID), lambda i: (i, 0)),
        out_shape=jax.ShapeDtypeStruct((EW8, HID), f32),
    )(gsd, gsd, M)


def _tc_final(parts, denp, xwr, rec, M, ba_r, P,
              C1W, C1b, C2W, C2b, C3W, C3b):
    def body(p_ref, d_ref, xwr_ref, rec_ref, m_ref, ba_ref, pm_ref,
             w1_ref, c1_ref, w2_ref, c2_ref, w3_ref, c3_ref,
             out_ref, acc_ref):
        i = pl.program_id(0)
        as4 = rec_ref[...][:, 0:4]
        ad4 = rec_ref[...][:, 4:8]
        m4 = m_ref[...][:, 0:4]
        ex_s = jnp.exp(_leaky(as4 + ad4) - _leaky(ad4 + m4))
        exb = jnp.concatenate([ex_s] * (HID // 4), axis=1)
        num = p_ref[0] + p_ref[1] + exb * xwr_ref[...]
        den4 = d_ref[0][:, 0:4] + d_ref[1][:, 0:4] + ex_s
        den = jnp.concatenate([den4] * (HID // 4), axis=1)
        h = jnp.maximum(num / (den + 1e-16) + ba_ref[...], 0.0)
        ridx = i * BN + lax.broadcasted_iota(i32, (BN, 1), 0)
        h = jnp.where(ridx < N, h, 0.0)
        bsum = jnp.sum(h, axis=0, keepdims=True)

        @pl.when(i == 0)
        def _():
            acc_ref[...] = jnp.zeros_like(acc_ref)

        acc_ref[0:1, :] = acc_ref[0:1, :] + bsum

        @pl.when(i == NB - 1)
        def _():
            pooled_r = acc_ref[0:1, :] * (1.0 / N)
            pooled = lax.dot_general(
                pooled_r, pm_ref[...], (((1,), (1,)), ((), ())),
                preferred_element_type=f32)
            z1 = jnp.maximum(
                jnp.dot(pooled, w1_ref[...], preferred_element_type=f32)
                + c1_ref[...], 0.0)
            z2 = jnp.maximum(
                jnp.dot(z1, w2_ref[...], preferred_element_type=f32)
                + c2_ref[...], 0.0)
            out_ref[...] = (jnp.dot(z2, w3_ref[...],
                                    preferred_element_type=f32) + c3_ref[...])

    return pl.pallas_call(
        body,
        grid=(NB,),
        in_specs=[
            pl.BlockSpec((NC, BN, HID), lambda i: (0, i, 0)),
            pl.BlockSpec((NC, BN, LANES), lambda i: (0, i, 0)),
            pl.BlockSpec((BN, HID), lambda i: (i, 0)),
            pl.BlockSpec((BN, LANES), lambda i: (i, 0)),
            pl.BlockSpec((1, LANES), lambda i: (0, 0)),
            pl.BlockSpec((1, HID), lambda i: (0, 0)),
            pl.BlockSpec((HID, HID), lambda i: (0, 0)),
            pl.BlockSpec((HID, HID // 2), lambda i: (0, 0)),
            pl.BlockSpec((1, HID // 2), lambda i: (0, 0)),
            pl.BlockSpec((HID // 2, HID // 4), lambda i: (0, 0)),
            pl.BlockSpec((1, HID // 4), lambda i: (0, 0)),
            pl.BlockSpec((HID // 4, 2), lambda i: (0, 0)),
            pl.BlockSpec((1, 2), lambda i: (0, 0)),
        ],
        out_specs=pl.BlockSpec((1, 2), lambda i: (0, 0)),
        out_shape=jax.ShapeDtypeStruct((1, 2), f32),
        scratch_shapes=[pltpu.VMEM((8, HID), f32)],
    )(parts, denp, xwr, rec, M, ba_r, P,
      C1W, C1b, C2W, C2b, C3W, C3b)


def kernel(x, edge_index, W1, b1, W2, b2, W3, b3, Wa, a_src, a_dst, ba,
           C1W, C1b, C2W, C2b, C3W, C3b):
    s = edge_index[0].astype(i32)
    d = edge_index[1].astype(i32)
    pad = jnp.full((EPAD - E,), N, i32)
    s_p = jnp.concatenate([s, pad])
    d_p = jnp.concatenate([d, pad])
    s_r = s_p.reshape(NW, CPW, CH)
    d_r = d_p.reshape(NW, CPW, CH)
    sd_r = jnp.stack([s_r, d_r], axis=1).reshape(NW, CPW2, CH)

    x_p = jnp.concatenate([x, jnp.zeros((NPAD - N, HID), f32)])
    ones16 = jnp.ones((CH, LANES), f32)
    z16 = jnp.zeros((NPAD, LANES), f32)
    z128 = jnp.zeros((NPAD, HID), f32)

    h_i = jnp.arange(HEADS, dtype=i32)
    c_i = jnp.arange(OC, dtype=i32)
    o_vec = (h_i[:, None] * OC + c_i[None, :]).reshape(-1)
    r_vec = (c_i[None, :] * HEADS + h_i[:, None]).reshape(-1)
    P = jnp.zeros((HID, HID), f32).at[o_vec, r_vec].set(1.0)
    Arec = (jnp.zeros((HID, LANES), f32)
            .at[o_vec, jnp.repeat(h_i, OC)].set(a_src.reshape(-1))
            .at[o_vec, 4 + jnp.repeat(h_i, OC)].set(a_dst.reshape(-1)))
    ba_r = (ba.reshape(1, HID) @ P)

    b1r = b1.reshape(1, HID)
    b2r = b2.reshape(1, HID)
    b3r = b3.reshape(1, HID // 2)
    C1br = C1b.reshape(1, HID // 2)
    C2br = C2b.reshape(1, HID // 4)
    C3br = C3b.reshape(1, 2)

    degp = _sc_deg()(d_r, ones16, z16)
    dinvf, y1 = _tc_pre(x_p, W1, degp)

    p1 = _sc_agg()(s_r, d_r, y1, z128)
    h1, y2 = _tc_layer1(p1, y1, dinvf, b1r, W2)

    p2 = _sc_agg()(s_r, d_r, y2, z128)
    (y3p,) = _tc_layer2(p2, y2, dinvf, b2r, h1, W3)

    p3 = _sc_agg()(s_r, d_r, y3p, z128)
    xwr, rec = _tc_layer3(p3, y3p, dinvf, b3r, Wa, P, Arec)

    M = _tc_maxrec(rec)
    gsd = _sc_recgather()(sd_r, rec)
    exw = _tc_exp(gsd, M)

    p4 = _sc_gat()(s_r, d_r, xwr, exw, z128)
    denp = _sc_den()(d_r, exw, z16)
    out = _tc_final(p4, denp, xwr, rec, M, ba_r, P,
                    C1W, C1br, C2W, C2br, C3W, C3br)
    return out


# speedup vs baseline: 1.0017x; 1.0017x over previous
import functools

import jax
import jax.numpy as jnp
from jax import lax
from jax.experimental import pallas as pl
from jax.experimental.pallas import tpu as pltpu
from jax.experimental.pallas import tpu_sc as plsc

N = 10000
E = 320000
HID = 128
HEADS = 4
OC = 32

NC, NS, LANES = 2, 16, 16
NW = NC * NS
CH = 128
NPAD = 10240
RPT = NPAD // NS
EPW = 10240
CPW = EPW // CH
EPAD = NW * EPW
CPW2 = 2 * CPW
IB = 16
EW8 = EPAD // 8
RPC = CH // 8


@functools.cache
def _mesh():
    return plsc.VectorSubcoreMesh(
        core_axis_name="c", subcore_axis_name="s",
        num_cores=NC, num_subcores=NS)

f32 = jnp.float32
i32 = jnp.int32


def _ids():
    cid = lax.axis_index("c")
    sid = lax.axis_index("s")
    return cid, sid, sid * NC + cid


def _zero_acc(z_hbm, acc, sid):
    r0 = sid * RPT
    pltpu.sync_copy(z_hbm.at[pl.ds(r0, RPT)], acc.at[pl.ds(r0, RPT)])
    plsc.subcore_barrier()


def _dump_acc(acc, out_hbm, cid, sid):
    plsc.subcore_barrier()
    r0 = sid * RPT
    pltpu.sync_copy(acc.at[pl.ds(r0, RPT)], out_hbm.at[cid, pl.ds(r0, RPT)])


def _sc_deg():
    def body(d_hbm, ones_hbm, z_hbm, out_hbm, didx, onesb, s0, s1, acc):
        cid, sid, wid = _ids()
        pltpu.sync_copy(ones_hbm, onesb)
        _zero_acc(z_hbm, acc, sid)
        ssems = (s0, s1)

        def blk(k, _):
            pltpu.sync_copy(d_hbm.at[wid, pl.ds(k * IB, IB)], didx)
            for b in range(2):
                pltpu.async_copy(onesb, acc.at[didx.at[b]], ssems[b],
                                 add=True)

            def step(g, _):
                for b in range(2):
                    c = g * 2 + b
                    pltpu.make_async_copy(
                        onesb, acc.at[didx.at[c]], ssems[b]).wait()

                    @pl.when(c + 2 < IB)
                    def _():
                        pltpu.async_copy(
                            onesb, acc.at[didx.at[c + 2]], ssems[b], add=True)
                return ()
            lax.fori_loop(0, IB // 2, step, ())
            return ()
        lax.fori_loop(0, CPW // IB, blk, ())
        _dump_acc(acc, out_hbm, cid, sid)

    return pl.kernel(
        body,
        out_type=jax.ShapeDtypeStruct((NC, NPAD, LANES), f32),
        mesh=_mesh(),
        scratch_types=[
            pltpu.VMEM((IB, CH), i32),
            pltpu.VMEM((CH, LANES), f32),
            pltpu.SemaphoreType.DMA,
            pltpu.SemaphoreType.DMA,
            pltpu.VMEM_SHARED((NPAD, LANES), f32),
        ],
    )


def _sc_agg():
    def body(s_hbm, d_hbm, y_hbm, z_hbm, out_hbm,
             sidx, didx, b0, b1, g0, g1, s0, s1, acc):
        cid, sid, wid = _ids()
        _zero_acc(z_hbm, acc, sid)

        bufs = (b0, b1)
        gsems = (g0, g1)
        ssems = (s0, s1)

        def blk(k, _):
            pltpu.sync_copy(s_hbm.at[wid, pl.ds(k * IB, IB)], sidx)
            pltpu.sync_copy(d_hbm.at[wid, pl.ds(k * IB, IB)], didx)
            for b in range(2):
                pltpu.async_copy(y_hbm.at[sidx.at[b]], bufs[b], gsems[b])

            def step(g, _):
                for b in range(2):
                    c = g * 2 + b
                    pltpu.make_async_copy(
                        y_hbm.at[sidx.at[c]], bufs[b], gsems[b]).wait()
                    pltpu.async_copy(
                        bufs[b], acc.at[didx.at[c]], ssems[b], add=True)
                    pltpu.make_async_copy(
                        bufs[b], acc.at[didx.at[c]], ssems[b]).wait()

                    @pl.when(c + 2 < IB)
                    def _():
                        pltpu.async_copy(
                            y_hbm.at[sidx.at[c + 2]], bufs[b], gsems[b])
                return ()
            lax.fori_loop(0, IB // 2, step, ())
            return ()
        lax.fori_loop(0, CPW // IB, blk, ())
        _dump_acc(acc, out_hbm, cid, sid)

    return pl.kernel(
        body,
        out_type=jax.ShapeDtypeStruct((NC, NPAD, HID), f32),
        mesh=_mesh(),
        scratch_types=[
            pltpu.VMEM((IB, CH), i32),
            pltpu.VMEM((IB, CH), i32),
            pltpu.VMEM((CH, HID), f32),
            pltpu.VMEM((CH, HID), f32),
            pltpu.SemaphoreType.DMA,
            pltpu.SemaphoreType.DMA,
            pltpu.SemaphoreType.DMA,
            pltpu.SemaphoreType.DMA,
            pltpu.VMEM_SHARED((NPAD, HID), f32),
        ],
    )


def _sc_recgather():
    def body(sd_hbm, rec_hbm, out_hbm, idx, b0, b1, p0, p1,
             g0, g1, w0, w1, tbl):
        cid, sid, wid = _ids()
        r0 = sid * RPT
        pltpu.sync_copy(rec_hbm.at[pl.ds(r0, RPT)], tbl.at[pl.ds(r0, RPT)])
        plsc.subcore_barrier()

        bufs = (b0, b1)
        pbufs = (p0, p1)
        gsems = (g0, g1)
        wsems = (w0, w1)

        def blk(k, _):
            pltpu.sync_copy(sd_hbm.at[wid, pl.ds(k * IB, IB)], idx)
            for b in range(2):
                pltpu.async_copy(tbl.at[idx.at[b]], bufs[b], gsems[b])

            def step(g, _):
                for b in range(2):
                    c = g * 2 + b
                    tg = k * IB + c
                    off = ((tg // CPW) * EW8 + wid * (EPW // 8)
                           + (tg % CPW) * RPC)
                    pltpu.make_async_copy(
                        tbl.at[idx.at[c]], bufs[b], gsems[b]).wait()

                    buf, pbuf = bufs[b], pbufs[b]
                    for r in range(RPC):
                        for q in range(8):
                            pbuf[r, pl.ds(q * 16, 16)] = (
                                buf[r * 8 + q, pl.ds(0, 16)])
                    pltpu.async_copy(
                        pbuf, out_hbm.at[pl.ds(off, RPC)], wsems[b])
                    pltpu.make_async_copy(
                        pbuf, out_hbm.at[pl.ds(off, RPC)], wsems[b]).wait()

                    @pl.when(c + 2 < IB)
                    def _():
                        pltpu.async_copy(
                            tbl.at[idx.at[c + 2]], bufs[b], gsems[b])
                return ()
            lax.fori_loop(0, IB // 2, step, ())
            return ()
        lax.fori_loop(0, CPW2 // IB, blk, ())

    return pl.kernel(
        body,
        out_type=jax.ShapeDtypeStruct((2 * EW8, HID), f32),
        mesh=_mesh(),
        scratch_types=[
            pltpu.VMEM((IB, CH), i32),
            pltpu.VMEM((CH, LANES), f32),
            pltpu.VMEM((CH, LANES), f32),
            pltpu.VMEM((RPC, HID), f32),
            pltpu.VMEM((RPC, HID), f32),
            pltpu.SemaphoreType.DMA,
            pltpu.SemaphoreType.DMA,
            pltpu.SemaphoreType.DMA,
            pltpu.SemaphoreType.DMA,
            pltpu.VMEM_SHARED((NPAD, LANES), f32),
        ],
    )


def _sc_gat():
    def body(s_hbm, d_hbm, xwr_hbm, exw_hbm, z_hbm, out_hbm,
             sidx, didx, gb0, gb1, eb0, eb1,
             g0, g1, s0, s1, e0, e1, acc):
        cid, sid, wid = _ids()
        _zero_acc(z_hbm, acc, sid)

        gbufs = (gb0, gb1)
        ebufs = (eb0, eb1)
        gsems = (g0, g1)
        ssems = (s0, s1)
        esems = (e0, e1)
        erow0 = wid * (EPW // 8)

        def blk(k, _):
            pltpu.sync_copy(s_hbm.at[wid, pl.ds(k * IB, IB)], sidx)
            pltpu.sync_copy(d_hbm.at[wid, pl.ds(k * IB, IB)], didx)
            for b in range(2):
                pltpu.async_copy(xwr_hbm.at[sidx.at[b]], gbufs[b], gsems[b])
                pltpu.async_copy(
                    exw_hbm.at[pl.ds(erow0 + (k * IB + b) * RPC, RPC)],
                    ebufs[b], esems[b])

            def step(g, _):
                for b in range(2):
                    c = g * 2 + b
                    pltpu.make_async_copy(
                        xwr_hbm.at[sidx.at[c]], gbufs[b], gsems[b]).wait()
                    pltpu.make_async_copy(
                        exw_hbm.at[pl.ds(erow0 + (k * IB + c) * RPC, RPC)],
                        ebufs[b], esems[b]).wait()

                    gbuf, ebuf = gbufs[b], ebufs[b]

                    def edge(i, _):
                        exv = ebuf[i >> 3, pl.ds((i & 7) * 16, 16)]
                        for j in range(HID // LANES):
                            gbuf[i, pl.ds(j * LANES, LANES)] = (
                                gbuf[i, pl.ds(j * LANES, LANES)] * exv)
                        return ()
                    lax.fori_loop(0, CH, edge, (), unroll=2)

                    pltpu.async_copy(
                        gbuf, acc.at[didx.at[c]], ssems[b], add=True)
                    pltpu.make_async_copy(
                        gbuf, acc.at[didx.at[c]], ssems[b]).wait()

                    @pl.when(c + 2 < IB)
                    def _():
                        pltpu.async_copy(
                            xwr_hbm.at[sidx.at[c + 2]], gbufs[b], gsems[b])
                        pltpu.async_copy(
                            exw_hbm.at[
                                pl.ds(erow0 + (k * IB + c + 2) * RPC, RPC)],
                            ebufs[b], esems[b])
                return ()
            lax.fori_loop(0, IB // 2, step, ())
            return ()
        lax.fori_loop(0, CPW // IB, blk, ())
        _dump_acc(acc, out_hbm, cid, sid)

    return pl.kernel(
        body,
        out_type=jax.ShapeDtypeStruct((NC, NPAD, HID), f32),
        mesh=_mesh(),
        scratch_types=[
            pltpu.VMEM((IB, CH), i32),
            pltpu.VMEM((IB, CH), i32),
            pltpu.VMEM((CH, HID), f32),
            pltpu.VMEM((CH, HID), f32),
            pltpu.VMEM((RPC, HID), f32),
            pltpu.VMEM((RPC, HID), f32),
            pltpu.SemaphoreType.DMA,
            pltpu.SemaphoreType.DMA,
            pltpu.SemaphoreType.DMA,
            pltpu.SemaphoreType.DMA,
            pltpu.SemaphoreType.DMA,
            pltpu.SemaphoreType.DMA,
            pltpu.VMEM_SHARED((NPAD, HID), f32),
        ],
    )


def _sc_den():
    def body(d_hbm, exw_hbm, z_hbm, out_hbm, didx, w0, w1, sb,
             l0, l1, s0, s1, acc):
        cid, sid, wid = _ids()
        _zero_acc(z_hbm, acc, sid)

        wbufs = (w0, w1)
        lsems = (l0, l1)
        ssems = (s0, s1)
        erow0 = wid * (EPW // 8)

        def blk(k, _):
            pltpu.sync_copy(d_hbm.at[wid, pl.ds(k * IB, IB)], didx)
            for b in range(2):
                pltpu.async_copy(
                    exw_hbm.at[pl.ds(erow0 + (k * IB + b) * RPC, RPC)],
                    wbufs[b], lsems[b])

            def step(g, _):
                for b in range(2):
                    c = g * 2 + b
                    pltpu.make_async_copy(
                        exw_hbm.at[pl.ds(erow0 + (k * IB + c) * RPC, RPC)],
                        wbufs[b], lsems[b]).wait()
                    wbuf = wbufs[b]
                    for r in range(RPC):
                        for q in range(8):
                            sb[r * 8 + q, pl.ds(0, 16)] = (
                                wbuf[r, pl.ds(q * 16, 16)])
                    pltpu.async_copy(
                        sb, acc.at[didx.at[c]], ssems[b], add=True)
                    pltpu.make_async_copy(
                        sb, acc.at[didx.at[c]], ssems[b]).wait()

                    @pl.when(c + 2 < IB)
                    def _():
                        pltpu.async_copy(
                            exw_hbm.at[
                                pl.ds(erow0 + (k * IB + c + 2) * RPC, RPC)],
                            wbufs[b], lsems[b])
                return ()
            lax.fori_loop(0, IB // 2, step, ())
            return ()
        lax.fori_loop(0, CPW // IB, blk, ())
        _dump_acc(acc, out_hbm, cid, sid)

    return pl.kernel(
        body,
        out_type=jax.ShapeDtypeStruct((NC, NPAD, LANES), f32),
        mesh=_mesh(),
        scratch_types=[
            pltpu.VMEM((IB, CH), i32),
            pltpu.VMEM((RPC, HID), f32),
            pltpu.VMEM((RPC, HID), f32),
            pltpu.VMEM((CH, LANES), f32),
            pltpu.SemaphoreType.DMA,
            pltpu.SemaphoreType.DMA,
            pltpu.SemaphoreType.DMA,
            pltpu.SemaphoreType.DMA,
            pltpu.VMEM_SHARED((NPAD, LANES), f32),
        ],
    )


BN = 1024
NB = NPAD // BN
BGE = 2048
NBG = EW8 // BGE


def _leaky(z):
    return jnp.maximum(z, 0.2 * z)


def _tc_pre(x_p, W1, degp):
    def body(x_ref, w_ref, dp_ref, dinv_ref, y_ref):
        d2 = dp_ref[0] + dp_ref[1]
        deg = d2[:, 0:1] + 1.0
        dinv = lax.rsqrt(deg)
        dinv_ref[...] = jnp.broadcast_to(dinv, (BN, HID))
        y_ref[...] = dinv * jnp.dot(x_ref[...], w_ref[...],
                                    preferred_element_type=f32)
    return pl.pallas_call(
        body,
        grid=(NB,),
        in_specs=[
            pl.BlockSpec((BN, HID), lambda i: (i, 0)),
            pl.BlockSpec((HID, HID), lambda i: (0, 0)),
            pl.BlockSpec((NC, BN, LANES), lambda i: (0, i, 0)),
        ],
        out_specs=[
            pl.BlockSpec((BN, HID), lambda i: (i, 0)),
            pl.BlockSpec((BN, HID), lambda i: (i, 0)),
        ],
        out_shape=[
            jax.ShapeDtypeStruct((NPAD, HID), f32),
            jax.ShapeDtypeStruct((NPAD, HID), f32),
        ],
    )(x_p, W1, degp)


def _tc_layer1(parts, y1, dinvf, b1, W2):
    def body(p_ref, y_ref, di_ref, b_ref, w_ref, h_ref, yn_ref):
        di = di_ref[...]
        h = jnp.maximum(
            di * (p_ref[0] + p_ref[1] + y_ref[...]) + b_ref[...], 0.0)
        h_ref[...] = h
        yn_ref[...] = di * jnp.dot(h, w_ref[...], preferred_element_type=f32)
    return pl.pallas_call(
        body,
        grid=(NB,),
        in_specs=[
            pl.BlockSpec((NC, BN, HID), lambda i: (0, i, 0)),
            pl.BlockSpec((BN, HID), lambda i: (i, 0)),
            pl.BlockSpec((BN, HID), lambda i: (i, 0)),
            pl.BlockSpec((1, HID), lambda i: (0, 0)),
            pl.BlockSpec((HID, HID), lambda i: (0, 0)),
        ],
        out_specs=[
            pl.BlockSpec((BN, HID), lambda i: (i, 0)),
            pl.BlockSpec((BN, HID), lambda i: (i, 0)),
        ],
        out_shape=[
            jax.ShapeDtypeStruct((NPAD, HID), f32),
            jax.ShapeDtypeStruct((NPAD, HID), f32),
        ],
    )(parts, y1, dinvf, b1, W2)


def _tc_layer2(parts, y2, dinvf, b2, h1, W3):
    def body(p_ref, y_ref, di_ref, b_ref, h1_ref, w_ref, yn_ref):
        di = di_ref[...]
        h = jnp.maximum(
            di * (p_ref[0] + p_ref[1] + y_ref[...]) + b_ref[...],
            0.0) + h1_ref[...]
        yn = di[:, : HID // 2] * jnp.dot(h, w_ref[...],
                                         preferred_element_type=f32)
        yn_ref[...] = jnp.concatenate(
            [yn, jnp.zeros((BN, HID // 2), f32)], axis=1)
    return pl.pallas_call(
        body,
        grid=(NB,),
        in_specs=[
            pl.BlockSpec((NC, BN, HID), lambda i: (0, i, 0)),
            pl.BlockSpec((BN, HID), lambda i: (i, 0)),
            pl.BlockSpec((BN, HID), lambda i: (i, 0)),
            pl.BlockSpec((1, HID), lambda i: (0, 0)),
            pl.BlockSpec((BN, HID), lambda i: (i, 0)),
            pl.BlockSpec((HID, HID // 2), lambda i: (0, 0)),
        ],
        out_specs=[pl.BlockSpec((BN, HID), lambda i: (i, 0))],
        out_shape=[jax.ShapeDtypeStruct((NPAD, HID), f32)],
    )(parts, y2, dinvf, b2, h1, W3)


def _tc_layer3(parts, y3p, dinvf, b3, Wa, P, Arec):
    def body(p_ref, y_ref, di_ref, b_ref, wa_ref, pm_ref, ar_ref,
             xwr_ref, rec_ref):
        H2 = HID // 2
        a = (p_ref[0] + p_ref[1] + y_ref[...])[:, :H2]
        di = di_ref[...][:, :H2]
        h = jnp.maximum(di * a + b_ref[...], 0.0)
        xw = jnp.dot(h, wa_ref[...], preferred_element_type=f32)
        xwr_ref[...] = jnp.dot(xw, pm_ref[...], preferred_element_type=f32)
        rec_ref[...] = jnp.dot(xw, ar_ref[...], preferred_element_type=f32)
    return pl.pallas_call(
        body,
        grid=(NB,),
        in_specs=[
            pl.BlockSpec((NC, BN, HID), lambda i: (0, i, 0)),
            pl.BlockSpec((BN, HID), lambda i: (i, 0)),
            pl.BlockSpec((BN, HID), lambda i: (i, 0)),
            pl.BlockSpec((1, HID // 2), lambda i: (0, 0)),
            pl.BlockSpec((HID // 2, HID), lambda i: (0, 0)),
            pl.BlockSpec((HID, HID), lambda i: (0, 0)),
            pl.BlockSpec((HID, LANES), lambda i: (0, 0)),
        ],
        out_specs=[
            pl.BlockSpec((BN, HID), lambda i: (i, 0)),
            pl.BlockSpec((BN, LANES), lambda i: (i, 0)),
        ],
        out_shape=[
            jax.ShapeDtypeStruct((NPAD, HID), f32),
            jax.ShapeDtypeStruct((NPAD, LANES), f32),
        ],
    )(parts, y3p, dinvf, b3, Wa, P, Arec)


def _tc_maxrec(rec):
    def body(r_ref, m_ref):
        i = pl.program_id(0)
        bm = jnp.max(r_ref[...], axis=0, keepdims=True)

        @pl.when(i == 0)
        def _():
            m_ref[...] = bm

        @pl.when(i > 0)
        def _():
            m_ref[...] = jnp.maximum(m_ref[...], bm)
    return pl.pallas_call(
        body,
        grid=(NB,),
        in_specs=[pl.BlockSpec((BN, LANES), lambda i: (i, 0))],
        out_specs=pl.BlockSpec((1, LANES), lambda i: (0, 0)),
        out_shape=jax.ShapeDtypeStruct((1, LANES), f32),
    )(rec)


def _tc_exp(gsd, M):
    def body(gs_ref, gd_ref, m_ref, ex_ref):
        gs = gs_ref[...]
        gd = gd_ref[...]
        m4 = m_ref[...][:, 0:4]
        cols = []
        for q in range(8):
            as4 = gs[:, 16 * q:16 * q + 4]
            ad4 = gd[:, 16 * q + 4:16 * q + 8]
            ex = jnp.exp(_leaky(as4 + ad4) - _leaky(ad4 + m4))
            cols.append(jnp.concatenate([ex, ex, ex, ex], axis=1))
        ex_ref[...] = jnp.concatenate(cols, axis=1)
    return pl.pallas_call(
        body,
        grid=(NBG,),
        in_specs=[
            pl.BlockSpec((BGE, HID), lambda i: (i, 0)),
            pl.BlockSpec((BGE, HID), lambda i: (i + NBG, 0)),
            pl.BlockSpec((1, LANES), lambda i: (0, 0)),
        ],
        out_specs=pl.BlockSpec((BGE, HID), lambda i: (i, 0)),
        out_shape=jax.ShapeDtypeStruct((EW8, HID), f32),
    )(gsd, gsd, M)


def _tc_final(parts, denp, xwr, rec, M, ba_r, P,
              C1W, C1b, C2W, C2b, C3W, C3b):
    def body(p_ref, d_ref, xwr_ref, rec_ref, m_ref, ba_ref, pm_ref,
             w1_ref, c1_ref, w2_ref, c2_ref, w3_ref, c3_ref,
             out_ref, acc_ref):
        i = pl.program_id(0)
        as4 = rec_ref[...][:, 0:4]
        ad4 = rec_ref[...][:, 4:8]
        m4 = m_ref[...][:, 0:4]
        ex_s = jnp.exp(_leaky(as4 + ad4) - _leaky(ad4 + m4))
        exb = jnp.concatenate([ex_s] * (HID // 4), axis=1)
        num = p_ref[0] + p_ref[1] + exb * xwr_ref[...]
        den4 = d_ref[0][:, 0:4] + d_ref[1][:, 0:4] + ex_s
        den = jnp.concatenate([den4] * (HID // 4), axis=1)
        h = jnp.maximum(num / (den + 1e-16) + ba_ref[...], 0.0)
        ridx = i * BN + lax.broadcasted_iota(i32, (BN, 1), 0)
        h = jnp.where(ridx < N, h, 0.0)
        bsum = jnp.sum(h, axis=0, keepdims=True)

        @pl.when(i == 0)
        def _():
            acc_ref[...] = jnp.zeros_like(acc_ref)

        acc_ref[0:1, :] = acc_ref[0:1, :] + bsum

        @pl.when(i == NB - 1)
        def _():
            pooled_r = acc_ref[0:1, :] * (1.0 / N)
            pooled = lax.dot_general(
                pooled_r, pm_ref[...], (((1,), (1,)), ((), ())),
                preferred_element_type=f32)
            z1 = jnp.maximum(
                jnp.dot(pooled, w1_ref[...], preferred_element_type=f32)
                + c1_ref[...], 0.0)
            z2 = jnp.maximum(
                jnp.dot(z1, w2_ref[...], preferred_element_type=f32)
                + c2_ref[...], 0.0)
            out_ref[...] = (jnp.dot(z2, w3_ref[...],
                                    preferred_element_type=f32) + c3_ref[...])

    return pl.pallas_call(
        body,
        grid=(NB,),
        in_specs=[
            pl.BlockSpec((NC, BN, HID), lambda i: (0, i, 0)),
            pl.BlockSpec((NC, BN, LANES), lambda i: (0, i, 0)),
            pl.BlockSpec((BN, HID), lambda i: (i, 0)),
            pl.BlockSpec((BN, LANES), lambda i: (i, 0)),
            pl.BlockSpec((1, LANES), lambda i: (0, 0)),
            pl.BlockSpec((1, HID), lambda i: (0, 0)),
            pl.BlockSpec((HID, HID), lambda i: (0, 0)),
            pl.BlockSpec((HID, HID // 2), lambda i: (0, 0)),
            pl.BlockSpec((1, HID // 2), lambda i: (0, 0)),
            pl.BlockSpec((HID // 2, HID // 4), lambda i: (0, 0)),
            pl.BlockSpec((1, HID // 4), lambda i: (0, 0)),
            pl.BlockSpec((HID // 4, 2), lambda i: (0, 0)),
            pl.BlockSpec((1, 2), lambda i: (0, 0)),
        ],
        out_specs=pl.BlockSpec((1, 2), lambda i: (0, 0)),
        out_shape=jax.ShapeDtypeStruct((1, 2), f32),
        scratch_shapes=[pltpu.VMEM((8, HID), f32)],
    )(parts, denp, xwr, rec, M, ba_r, P,
      C1W, C1b, C2W, C2b, C3W, C3b)


def kernel(x, edge_index, W1, b1, W2, b2, W3, b3, Wa, a_src, a_dst, ba,
           C1W, C1b, C2W, C2b, C3W, C3b):
    s = edge_index[0].astype(i32)
    d = edge_index[1].astype(i32)
    pad = jnp.full((EPAD - E,), N, i32)
    s_p = jnp.concatenate([s, pad])
    d_p = jnp.concatenate([d, pad])
    s_r = s_p.reshape(NW, CPW, CH)
    d_r = d_p.reshape(NW, CPW, CH)
    sd_r = jnp.stack([s_r, d_r], axis=1).reshape(NW, CPW2, CH)

    x_p = jnp.concatenate([x, jnp.zeros((NPAD - N, HID), f32)])
    ones16 = jnp.ones((CH, LANES), f32)
    z16 = jnp.zeros((NPAD, LANES), f32)
    z128 = jnp.zeros((NPAD, HID), f32)

    h_i = jnp.arange(HEADS, dtype=i32)
    c_i = jnp.arange(OC, dtype=i32)
    o_vec = (h_i[:, None] * OC + c_i[None, :]).reshape(-1)
    r_vec = (c_i[None, :] * HEADS + h_i[:, None]).reshape(-1)
    P = jnp.zeros((HID, HID), f32).at[o_vec, r_vec].set(1.0)
    Arec = (jnp.zeros((HID, LANES), f32)
            .at[o_vec, jnp.repeat(h_i, OC)].set(a_src.reshape(-1))
            .at[o_vec, 4 + jnp.repeat(h_i, OC)].set(a_dst.reshape(-1)))
    ba_r = (ba.reshape(1, HID) @ P)

    b1r = b1.reshape(1, HID)
    b2r = b2.reshape(1, HID)
    b3r = b3.reshape(1, HID // 2)
    C1br = C1b.reshape(1, HID // 2)
    C2br = C2b.reshape(1, HID // 4)
    C3br = C3b.reshape(1, 2)

    degp = _sc_deg()(d_r, ones16, z16)
    dinvf, y1 = _tc_pre(x_p, W1, degp)

    p1 = _sc_agg()(s_r, d_r, y1, z128)
    h1, y2 = _tc_layer1(p1, y1, dinvf, b1r, W2)

    p2 = _sc_agg()(s_r, d_r, y2, z128)
    (y3p,) = _tc_layer2(p2, y2, dinvf, b2r, h1, W3)

    p3 = _sc_agg()(s_r, d_r, y3p, z128)
    xwr, rec = _tc_layer3(p3, y3p, dinvf, b3r, Wa, P, Arec)

    M = _tc_maxrec(rec)
    gsd = _sc_recgather()(sd_r, rec)
    exw = _tc_exp(gsd, M)

    p4 = _sc_gat()(s_r, d_r, xwr, exw, z128)
    denp = _sc_den()(d_r, exw, z16)
    out = _tc_final(p4, denp, xwr, rec, M, ba_r, P,
                    C1W, C1br, C2W, C2br, C3W, C3br)
    return out
